# baseline (speedup 1.0000x reference)
import functools

import numpy as np
import jax
import jax.numpy as jnp
from jax import lax
from jax.experimental import pallas as pl
from jax.experimental.pallas import tpu as pltpu
from jax.experimental.pallas import tpu_sc as plsc

_MAX_LEN = 2048
_NC = 2
_NS = 16
_NW = _NC * _NS
_L = 16


def _pos_table_packed(max_len, E):
    pos = np.arange(max_len, dtype=np.float64)[:, None]
    i = np.arange(E, dtype=np.float64)[None, :]
    angle = pos / np.power(10000.0, (i - np.mod(i, 2)) / E)
    angle[:, 0::2] = np.sin(angle[:, 0::2])
    angle[:, 1::2] = np.cos(angle[:, 1::2])
    flat = angle.reshape(-1).astype(np.float32)
    bits = flat.view(np.uint32)
    rounded = (bits + 0x7FFF + ((bits >> 16) & 1)) >> 16
    blocks = rounded.reshape(-1, 2, _L)
    words = blocks[:, 0, :] | (blocks[:, 1, :] << np.uint32(16))
    return jnp.asarray(words.reshape(-1).view(np.int32))


def _sc_call(x, pos_packed):
    B, S, E = x.shape
    ROWS = S // _NW
    EW = E // 2
    CH = 16
    NCH = ROWS // CH
    NBUF = 2
    UNROLL = 1

    mesh = plsc.VectorSubcoreMesh(core_axis_name="c", subcore_axis_name="s")

    @functools.partial(
        pl.kernel,
        mesh=mesh,
        out_type=jax.ShapeDtypeStruct((B, S, E), jnp.float32),
        scratch_types=(
            [pltpu.VMEM((ROWS * EW,), jnp.int32)]
            + [pltpu.VMEM((CH, E), jnp.float32)] * NBUF
            + [pltpu.VMEM((CH, E), jnp.float32)] * NBUF
            + [pltpu.SemaphoreType.DMA] * (2 * NBUF + 1)
        ),
    )
    def k(x_hbm, pos_hbm, out_hbm, pos_v, *bufs):
        xin = bufs[0:NBUF]
        xout = bufs[NBUF:2 * NBUF]
        sem_i = bufs[2 * NBUF:3 * NBUF]
        sem_o = bufs[3 * NBUF:4 * NBUF]
        sem_pos = bufs[4 * NBUF]

        wid = lax.axis_index("s") * _NC + lax.axis_index("c")
        base = wid * ROWS

        pos_dma = pltpu.async_copy(
            pos_hbm.at[pl.ds(base * EW, ROWS * EW)], pos_v, sem_pos)

        NT = B * NCH
        himask = jnp.full((_L,), -65536, dtype=jnp.int32)

        def chunk_ref(hbm, t):
            b, kch = divmod(t, NCH)
            return hbm.at[b, pl.ds(base + kch * CH, CH), :]

        in_dma = [None] * NT
        out_dma = [None] * NT
        for t in range(min(NBUF, NT)):
            in_dma[t] = pltpu.async_copy(
                chunk_ref(x_hbm, t), xin[t % NBUF], sem_i[t % NBUF])
        pos_dma.wait()

        for t in range(NT):
            p = t % NBUF
            in_dma[t].wait()
            if t >= NBUF:
                out_dma[t - NBUF].wait()
            r0 = (t % NCH) * CH
            xi, xo = xin[p], xout[p]

            @plsc.parallel_loop(0, E, step=2 * _L, unroll=UNROLL)
            def vec_body(i, r0=r0, xi=xi, xo=xo):
                iw = lax.div(i, 2)
                for r in range(CH):
                    pw = pos_v[pl.ds((r0 + r) * EW + iw, _L)]
                    pv0 = lax.bitcast_convert_type(
                        jnp.left_shift(pw, 16), jnp.float32)
                    pv1 = lax.bitcast_convert_type(
                        jnp.bitwise_and(pw, himask), jnp.float32)
                    xv0 = xi[r, pl.ds(i, _L)]
                    xv1 = xi[r, pl.ds(i + _L, _L)]
                    xo[r, pl.ds(i, _L)] = jnp.where(xv0 == 0.0, 0.0, pv0)
                    xo[r, pl.ds(i + _L, _L)] = jnp.where(xv1 == 0.0, 0.0, pv1)

            out_dma[t] = pltpu.async_copy(xo, chunk_ref(out_hbm, t), sem_o[p])
            if t + NBUF < NT:
                in_dma[t + NBUF] = pltpu.async_copy(
                    chunk_ref(x_hbm, t + NBUF), xi, sem_i[p])

        for t in range(max(0, NT - NBUF), NT):
            out_dma[t].wait()

    return k(x, pos_packed)


def kernel(x):
    B, S, E = x.shape
    assert S % _NW == 0 and E % (2 * _L) == 0
    pos_packed = _pos_table_packed(_MAX_LEN, E)[:S * (E // 2)]
    return _sc_call(x, pos_packed)


# speedup vs baseline: 1.0399x; 1.0399x over previous
import functools

import numpy as np
import jax
import jax.numpy as jnp
from jax import lax
from jax.experimental import pallas as pl
from jax.experimental.pallas import tpu as pltpu
from jax.experimental.pallas import tpu_sc as plsc

_MAX_LEN = 2048
_NC = 2
_NS = 16
_NW = _NC * _NS
_L = 16


def _pos_table_packed(max_len, E):
    pos = np.arange(max_len, dtype=np.float64)[:, None]
    i = np.arange(E, dtype=np.float64)[None, :]
    angle = pos / np.power(10000.0, (i - np.mod(i, 2)) / E)
    angle[:, 0::2] = np.sin(angle[:, 0::2])
    angle[:, 1::2] = np.cos(angle[:, 1::2])
    flat = angle.reshape(-1).astype(np.float32)
    bits = flat.view(np.uint32)
    rounded = (bits + 0x7FFF + ((bits >> 16) & 1)) >> 16
    blocks = rounded.reshape(-1, 2, _L)
    words = blocks[:, 0, :] | (blocks[:, 1, :] << np.uint32(16))
    return jnp.asarray(words.reshape(-1).view(np.int32))


def _sc_call(x, pos_packed):
    B, S, E = x.shape
    ROWS = S // _NW
    EW = E // 2
    CH = 8
    NCH = ROWS // CH
    NBUF = 4
    UNROLL = 1

    mesh = plsc.VectorSubcoreMesh(core_axis_name="c", subcore_axis_name="s")

    @functools.partial(
        pl.kernel,
        mesh=mesh,
        out_type=jax.ShapeDtypeStruct((B, S, E), jnp.float32),
        scratch_types=(
            [pltpu.VMEM((ROWS * EW,), jnp.int32)]
            + [pltpu.VMEM((CH, E), jnp.float32)] * NBUF
            + [pltpu.VMEM((CH, E), jnp.float32)] * NBUF
            + [pltpu.SemaphoreType.DMA] * (2 * NBUF + 1)
        ),
    )
    def k(x_hbm, pos_hbm, out_hbm, pos_v, *bufs):
        xin = bufs[0:NBUF]
        xout = bufs[NBUF:2 * NBUF]
        sem_i = bufs[2 * NBUF:3 * NBUF]
        sem_o = bufs[3 * NBUF:4 * NBUF]
        sem_pos = bufs[4 * NBUF]

        wid = lax.axis_index("s") * _NC + lax.axis_index("c")
        base = wid * ROWS

        pos_dma = pltpu.async_copy(
            pos_hbm.at[pl.ds(base * EW, ROWS * EW)], pos_v, sem_pos)

        NT = B * NCH
        himask = jnp.full((_L,), -65536, dtype=jnp.int32)

        def chunk_ref(hbm, t):
            b, kch = divmod(t, NCH)
            return hbm.at[b, pl.ds(base + kch * CH, CH), :]

        in_dma = [None] * NT
        out_dma = [None] * NT
        for t in range(min(NBUF, NT)):
            in_dma[t] = pltpu.async_copy(
                chunk_ref(x_hbm, t), xin[t % NBUF], sem_i[t % NBUF])
        pos_dma.wait()

        for t in range(NT):
            p = t % NBUF
            in_dma[t].wait()
            if t >= NBUF:
                out_dma[t - NBUF].wait()
            r0 = (t % NCH) * CH
            xi, xo = xin[p], xout[p]

            @plsc.parallel_loop(0, E, step=2 * _L, unroll=UNROLL)
            def vec_body(i, r0=r0, xi=xi, xo=xo):
                iw = lax.div(i, 2)
                for r in range(CH):
                    pw = pos_v[pl.ds((r0 + r) * EW + iw, _L)]
                    pv0 = lax.bitcast_convert_type(
                        jnp.left_shift(pw, 16), jnp.float32)
                    pv1 = lax.bitcast_convert_type(
                        jnp.bitwise_and(pw, himask), jnp.float32)
                    xv0 = xi[r, pl.ds(i, _L)]
                    xv1 = xi[r, pl.ds(i + _L, _L)]
                    xo[r, pl.ds(i, _L)] = jnp.where(xv0 == 0.0, 0.0, pv0)
                    xo[r, pl.ds(i + _L, _L)] = jnp.where(xv1 == 0.0, 0.0, pv1)

            out_dma[t] = pltpu.async_copy(xo, chunk_ref(out_hbm, t), sem_o[p])
            if t + NBUF < NT:
                in_dma[t + NBUF] = pltpu.async_copy(
                    chunk_ref(x_hbm, t + NBUF), xi, sem_i[p])

        for t in range(max(0, NT - NBUF), NT):
            out_dma[t].wait()

    return k(x, pos_packed)


def kernel(x):
    B, S, E = x.shape
    assert S % _NW == 0 and E % (2 * _L) == 0
    pos_packed = _pos_table_packed(_MAX_LEN, E)[:S * (E // 2)]
    return _sc_call(x, pos_packed)


# speedup vs baseline: 1.0509x; 1.0106x over previous
import functools

import numpy as np
import jax
import jax.numpy as jnp
from jax import lax
from jax.experimental import pallas as pl
from jax.experimental.pallas import tpu as pltpu
from jax.experimental.pallas import tpu_sc as plsc

_MAX_LEN = 2048
_NC = 2
_NS = 16
_NW = _NC * _NS
_L = 16


def _pos_table_packed(max_len, E):
    pos = np.arange(max_len, dtype=np.float64)[:, None]
    i = np.arange(E, dtype=np.float64)[None, :]
    angle = pos / np.power(10000.0, (i - np.mod(i, 2)) / E)
    angle[:, 0::2] = np.sin(angle[:, 0::2])
    angle[:, 1::2] = np.cos(angle[:, 1::2])
    flat = angle.reshape(-1).astype(np.float32)
    bits = flat.view(np.uint32)
    rounded = (bits + 0x7FFF + ((bits >> 16) & 1)) >> 16
    blocks = rounded.reshape(-1, 2, _L)
    words = blocks[:, 0, :] | (blocks[:, 1, :] << np.uint32(16))
    return jnp.asarray(words.reshape(-1).view(np.int32))


def _sc_call(x, pos_packed):
    B, S, E = x.shape
    ROWS = S // _NW
    EW = E // 2
    CH = 8
    NCH = ROWS // CH
    NBUF = 5
    UNROLL = 1

    mesh = plsc.VectorSubcoreMesh(core_axis_name="c", subcore_axis_name="s")

    @functools.partial(
        pl.kernel,
        mesh=mesh,
        out_type=jax.ShapeDtypeStruct((B, S, E), jnp.float32),
        scratch_types=(
            [pltpu.VMEM((ROWS * EW,), jnp.int32)]
            + [pltpu.VMEM((CH, E), jnp.float32)] * NBUF
            + [pltpu.VMEM((CH, E), jnp.float32)] * NBUF
            + [pltpu.SemaphoreType.DMA] * (2 * NBUF + 1)
        ),
    )
    def k(x_hbm, pos_hbm, out_hbm, pos_v, *bufs):
        xin = bufs[0:NBUF]
        xout = bufs[NBUF:2 * NBUF]
        sem_i = bufs[2 * NBUF:3 * NBUF]
        sem_o = bufs[3 * NBUF:4 * NBUF]
        sem_pos = bufs[4 * NBUF]

        wid = lax.axis_index("s") * _NC + lax.axis_index("c")
        base = wid * ROWS

        pos_dma = pltpu.async_copy(
            pos_hbm.at[pl.ds(base * EW, ROWS * EW)], pos_v, sem_pos)

        NT = B * NCH
        himask = jnp.full((_L,), -65536, dtype=jnp.int32)

        def chunk_ref(hbm, t):
            b, kch = divmod(t, NCH)
            return hbm.at[b, pl.ds(base + kch * CH, CH), :]

        in_dma = [None] * NT
        out_dma = [None] * NT
        for t in range(min(NBUF, NT)):
            in_dma[t] = pltpu.async_copy(
                chunk_ref(x_hbm, t), xin[t % NBUF], sem_i[t % NBUF])
        pos_dma.wait()

        for t in range(NT):
            p = t % NBUF
            in_dma[t].wait()
            if t >= NBUF:
                out_dma[t - NBUF].wait()
            r0 = (t % NCH) * CH
            xi, xo = xin[p], xout[p]

            @plsc.parallel_loop(0, E, step=2 * _L, unroll=UNROLL)
            def vec_body(i, r0=r0, xi=xi, xo=xo):
                iw = lax.div(i, 2)
                for r in range(CH):
                    pw = pos_v[pl.ds((r0 + r) * EW + iw, _L)]
                    pv0 = lax.bitcast_convert_type(
                        jnp.left_shift(pw, 16), jnp.float32)
                    pv1 = lax.bitcast_convert_type(
                        jnp.bitwise_and(pw, himask), jnp.float32)
                    xv0 = xi[r, pl.ds(i, _L)]
                    xv1 = xi[r, pl.ds(i + _L, _L)]
                    xo[r, pl.ds(i, _L)] = jnp.where(xv0 == 0.0, 0.0, pv0)
                    xo[r, pl.ds(i + _L, _L)] = jnp.where(xv1 == 0.0, 0.0, pv1)

            out_dma[t] = pltpu.async_copy(xo, chunk_ref(out_hbm, t), sem_o[p])
            if t + NBUF < NT:
                in_dma[t + NBUF] = pltpu.async_copy(
                    chunk_ref(x_hbm, t + NBUF), xi, sem_i[p])

        for t in range(max(0, NT - NBUF), NT):
            out_dma[t].wait()

    return k(x, pos_packed)


def kernel(x):
    B, S, E = x.shape
    assert S % _NW == 0 and E % (2 * _L) == 0
    pos_packed = _pos_table_packed(_MAX_LEN, E)[:S * (E // 2)]
    return _sc_call(x, pos_packed)
